# baseline (speedup 1.0000x reference)
import functools
import math

import jax
import jax.numpy as jnp
from jax import lax
from jax.experimental import pallas as pl
from jax.experimental.pallas import tpu as pltpu
from jax.experimental.pallas import tpu_sc as plsc

N = 10000
E = 160000
MUL = 16
HID = 64
NR = MUL * MUL

NC = 2
NS = 16
NW = NC * NS

IB = 100
CH = 200
KPW = 25
EPW = KPW * CH
XR = 1280
XRPS = XR // NS


@functools.lru_cache(maxsize=1)
def _sc_kernels():
    mesh = plsc.VectorSubcoreMesh(core_axis_name="c", subcore_axis_name="s")

    @functools.partial(
        pl.kernel,
        mesh=mesh,
        out_type=jax.ShapeDtypeStruct((E, 128), jnp.float32),
        scratch_types=[
            pltpu.VMEM_SHARED((XR, 128), jnp.float32),
            pltpu.VMEM((2 * KPW, IB), jnp.int32),
            pltpu.VMEM((2, CH, 128), jnp.float32),
            pltpu.SemaphoreType.DMA,
            pltpu.SemaphoreType.DMA,
        ],
    )
    def sc_gather(x_hbm, src8_hbm, out_hbm, x_sh, idx8_v, chunk_v, gsem, osem):
        cid = lax.axis_index("c")
        sid = lax.axis_index("s")
        wid = sid * NC + cid
        pltpu.sync_copy(x_hbm.at[pl.ds(sid * XRPS, XRPS)],
                        x_sh.at[pl.ds(sid * XRPS, XRPS)])
        pltpu.sync_copy(src8_hbm.at[wid], idx8_v)
        plsc.subcore_barrier()

        def start(j, slot):
            pltpu.async_copy(x_sh.at[idx8_v.at[2 * j]],
                             chunk_v.at[slot].at[pl.ds(0, IB)], gsem)
            pltpu.async_copy(x_sh.at[idx8_v.at[2 * j + 1]],
                             chunk_v.at[slot].at[pl.ds(IB, IB)], gsem)

        def wait(j, slot):
            pltpu.make_async_copy(x_sh.at[idx8_v.at[2 * j]],
                                  chunk_v.at[slot].at[pl.ds(0, IB)], gsem).wait()
            pltpu.make_async_copy(x_sh.at[idx8_v.at[2 * j + 1]],
                                  chunk_v.at[slot].at[pl.ds(IB, IB)], gsem).wait()

        start(0, 0)

        def body(j, carry):
            slot = lax.rem(j, 2)
            wait(j, slot)

            @pl.when(j + 1 < KPW)
            def _():
                start(j + 1, 1 - slot)

            ob = pl.multiple_of(wid * EPW + j * CH, 8)
            pltpu.async_copy(chunk_v.at[slot], out_hbm.at[pl.ds(ob, CH)],
                             osem).wait()
            return carry

        lax.fori_loop(0, KPW, body, 0)

    @functools.partial(
        pl.kernel,
        mesh=mesh,
        out_type=jax.ShapeDtypeStruct((NC, XR, 128), jnp.float32),
        scratch_types=[
            pltpu.VMEM_SHARED((XR, 128), jnp.float32),
            pltpu.VMEM((2 * KPW, IB), jnp.int32),
            pltpu.VMEM((2, CH, 128), jnp.float32),
            pltpu.SemaphoreType.DMA,
        ],
    )
    def sc_scatter(msg_hbm, dst8_hbm, zero_hbm, out_hbm,
                   acc_sh, idx8_v, mbuf_v, msem):
        cid = lax.axis_index("c")
        sid = lax.axis_index("s")
        wid = sid * NC + cid
        pltpu.sync_copy(dst8_hbm.at[wid], idx8_v)
        pltpu.sync_copy(zero_hbm.at[pl.ds(sid * XRPS, XRPS)],
                        acc_sh.at[pl.ds(sid * XRPS, XRPS)])
        plsc.subcore_barrier()

        def start(j, slot):
            mb = pl.multiple_of(wid * EPW + j * CH, 8)
            pltpu.async_copy(msg_hbm.at[pl.ds(mb, CH)], mbuf_v.at[slot], msem)

        def wait(j, slot):
            mb = pl.multiple_of(wid * EPW + j * CH, 8)
            pltpu.make_async_copy(msg_hbm.at[pl.ds(mb, CH)],
                                  mbuf_v.at[slot], msem).wait()

        start(0, 0)

        def body(j, carry):
            slot = lax.rem(j, 2)
            wait(j, slot)

            @pl.when(j + 1 < KPW)
            def _():
                start(j + 1, 1 - slot)

            pltpu.sync_copy(mbuf_v.at[slot].at[pl.ds(0, IB)],
                            acc_sh.at[idx8_v.at[2 * j]], add=True)
            pltpu.sync_copy(mbuf_v.at[slot].at[pl.ds(IB, IB)],
                            acc_sh.at[idx8_v.at[2 * j + 1]], add=True)
            return carry

        lax.fori_loop(0, KPW, body, 0)
        plsc.subcore_barrier()
        pltpu.sync_copy(acc_sh.at[pl.ds(sid * XRPS, XRPS)],
                        out_hbm.at[cid, pl.ds(sid * XRPS, XRPS)])

    return sc_gather, sc_scatter


TE = 4000
GRID = E // TE


def _tc_body(d_ref, f_ref, sg_ref, dg_ref, w1_ref, b1_ref, w2_ref, b2_ref,
             tt_ref, s_ref, t8_ref, o_ref):
    lanegrp = lax.broadcasted_iota(jnp.int32, (TE, 128), 1) >> 4
    fmask = jnp.where(lanegrp == sg_ref[...], f_ref[...], 0.0)
    h = jnp.maximum(d_ref[...] * w1_ref[...] + b1_ref[...], 0.0)
    r = jnp.dot(h, w2_ref[...], preferred_element_type=jnp.float32) + b2_ref[...]
    ft = jnp.dot(fmask, tt_ref[...], preferred_element_type=jnp.float32)
    msg = jnp.dot(r * ft, s_ref[...], preferred_element_type=jnp.float32)
    mp = jnp.dot(msg, t8_ref[...], preferred_element_type=jnp.float32)
    o_ref[...] = jnp.where(lanegrp == dg_ref[...], mp, 0.0)


def _tc_messages(d2, f128, sg2, dg2, w1, b1v, w2c, b2c, tt, s, t8):
    return pl.pallas_call(
        _tc_body,
        grid=(GRID,),
        in_specs=[
            pl.BlockSpec((TE, 1), lambda i: (i, 0)),
            pl.BlockSpec((TE, 128), lambda i: (i, 0)),
            pl.BlockSpec((TE, 1), lambda i: (i, 0)),
            pl.BlockSpec((TE, 1), lambda i: (i, 0)),
            pl.BlockSpec((1, HID), lambda i: (0, 0)),
            pl.BlockSpec((1, HID), lambda i: (0, 0)),
            pl.BlockSpec((HID, NR), lambda i: (0, 0)),
            pl.BlockSpec((1, NR), lambda i: (0, 0)),
            pl.BlockSpec((128, NR), lambda i: (0, 0)),
            pl.BlockSpec((NR, MUL), lambda i: (0, 0)),
            pl.BlockSpec((MUL, 128), lambda i: (0, 0)),
        ],
        out_specs=pl.BlockSpec((TE, 128), lambda i: (i, 0)),
        out_shape=jax.ShapeDtypeStruct((E, 128), jnp.float32),
    )(d2, f128, sg2, dg2, w1, b1v, w2c, b2c, tt, s, t8)


def kernel(x, edge_index, abs_distances, rel_vec, W1, b1, W2, b2):
    src = edge_index[0]
    dst = edge_index[1]

    src8 = (src >> 3).reshape(NW, 2 * KPW, IB)
    dst8 = (dst >> 3).reshape(NW, 2 * KPW, IB)
    sg2 = (src & 7).reshape(E, 1)
    dg2 = (dst & 7).reshape(E, 1)

    x2d = jnp.pad(x.reshape(N // 8, 128), ((0, XR - N // 8), (0, 0)))

    c = (math.sqrt(4.0 * math.pi) / math.sqrt(MUL)) * (1.0 / (2.0 * math.sqrt(math.pi)))
    w2c = W2 * c
    b2c = (b2 * c).reshape(1, NR)
    w1 = W1.reshape(1, HID)
    b1v = b1.reshape(1, HID)
    eye = jnp.eye(MUL, dtype=jnp.float32)
    tt = jnp.tile(jnp.tile(eye, (1, MUL)), (8, 1))
    s = jnp.repeat(eye, MUL, axis=0)
    t8 = jnp.tile(eye, (1, 8))

    sc_gather, sc_scatter = _sc_kernels()
    f128 = sc_gather(x2d, src8)
    msgp = _tc_messages(abs_distances.reshape(E, 1), f128, sg2, dg2,
                        w1, b1v, w2c, b2c, tt, s, t8)
    zero = jnp.zeros((XR, 128), dtype=jnp.float32)
    parts = sc_scatter(msgp, dst8, zero)
    acc = (parts[0] + parts[1]).reshape(XR * 8, MUL)
    return acc[:N]
